# baseline (device time: 64543 ns/iter reference)
import functools
import os

import jax
import jax.numpy as jnp
from jax import lax
from jax.experimental import pallas as pl
from jax.experimental.pallas import tpu as pltpu

B, SQ, H, D = 8, 8, 16, 128
SKV_LOCAL = 1024
HALF = SKV_LOCAL // 2
NSLOT = 3
SCALE = D ** -0.5

_KVAR = os.environ.get("KVAR", "full")


def kernel(Q, K, V):
    def body(q_ref, k_hbm, v_hbm, o_ref,
             k_buf, v_buf, o_acc, o_recv, st_acc, st_recv,
             copy_sems, send_sems, recv_sems):
        my_x = lax.axis_index("x")
        my_y = lax.axis_index("y")
        peers = ((my_x, 1 - my_y), (1 - my_x, my_y), (1 - my_x, 1 - my_y))

        barrier = pltpu.get_barrier_semaphore()
        for peer in peers:
            pl.semaphore_signal(barrier, inc=1, device_id=peer,
                                device_id_type=pl.DeviceIdType.MESH)
        pl.semaphore_wait(barrier, len(peers))

        row0 = my_x * HALF

        def dma_batch(b, slot):
            cps = []
            for h in range(H):
                cps.append(pltpu.make_async_copy(
                    k_hbm.at[b, pl.ds(row0, HALF), h],
                    k_buf.at[slot, h], copy_sems.at[slot, 0, h]))
                cps.append(pltpu.make_async_copy(
                    v_hbm.at[b, pl.ds(row0, HALF), h],
                    v_buf.at[slot, h], copy_sems.at[slot, 1, h]))
            for c in cps:
                c.start()
            return cps

        def exchange(phase, b):
            o_rdma = pltpu.make_async_remote_copy(
                src_ref=o_acc.at[b], dst_ref=o_recv.at[phase, b],
                send_sem=send_sems.at[phase, b, 0],
                recv_sem=recv_sems.at[phase, b, 0],
                device_id=peers[phase], device_id_type=pl.DeviceIdType.MESH)
            st_rdma = pltpu.make_async_remote_copy(
                src_ref=st_acc.at[b], dst_ref=st_recv.at[phase, b],
                send_sem=send_sems.at[phase, b, 1],
                recv_sem=recv_sems.at[phase, b, 1],
                device_id=peers[phase], device_id_type=pl.DeviceIdType.MESH)
            o_rdma.start()
            st_rdma.start()
            return (o_rdma, st_rdma)

        def combine_all(b, rdma_sets):
            for rdmas in rdma_sets:
                for r in rdmas:
                    r.wait()
            m_n = st_acc[b, 0]
            for p in range(3):
                m_n = jnp.maximum(m_n, st_recv[p, b, 0])
            a = jnp.exp(st_acc[b, 0] - m_n)
            l_n = a * st_acc[b, 1]
            o_n = a[..., None] * o_acc[b]
            for p in range(3):
                w = jnp.exp(st_recv[p, b, 0] - m_n)
                l_n = l_n + w * st_recv[p, b, 1]
                o_n = o_n + w[..., None] * o_recv[p, b]
            o_ref[b] = o_n / l_n[..., None]

        do_comm = _KVAR != "nocomm"
        inflight = {}

        pend = {0: dma_batch(0, 0), 1: dma_batch(1, 1)}
        for b in range(B):
            cps = pend.pop(b)
            if b + 2 < B:
                pend[b + 2] = dma_batch(b + 2, (b + 2) % NSLOT)
            slot = b % NSLOT
            ms = []
            ls = []
            if _KVAR == "nocompute":
                for c in cps:
                    c.wait()
            for h in range(H if _KVAR != "nocompute" else 0):
                cps[2 * h].wait()
                cps[2 * h + 1].wait()
                q_h = q_ref[b, :, h, :]
                k_h = k_buf[slot, h]
                v_h = v_buf[slot, h]
                s = lax.dot_general(
                    q_h, k_h, (((1,), (1,)), ((), ())),
                    preferred_element_type=jnp.float32) * SCALE
                m = jnp.max(s, axis=-1)
                p = jnp.exp(s - m[:, None])
                l = jnp.sum(p, axis=-1)
                o_h = lax.dot_general(
                    p, v_h, (((1,), (0,)), ((), ())),
                    preferred_element_type=jnp.float32)
                o_acc[b, :, h, :] = o_h
                ms.append(m)
                ls.append(l)
            if ms:
                st_acc[b, 0] = jnp.stack(ms, axis=1)
                st_acc[b, 1] = jnp.stack(ls, axis=1)

            if do_comm:
                inflight[b] = [exchange(p, b) for p in range(3)]
                if b >= 1:
                    combine_all(b - 1, inflight.pop(b - 1))

        if do_comm:
            combine_all(B - 1, inflight.pop(B - 1))
        else:
            for b in range(B):
                o_ref[b] = o_acc[b] / st_acc[b, 1][..., None]

        @functools.partial(pl.run_scoped, sem=pltpu.SemaphoreType.REGULAR)
        def _(sem):
            for peer in peers:
                pl.semaphore_signal(sem, inc=1, device_id=peer,
                                    device_id_type=pl.DeviceIdType.MESH)
            pl.semaphore_wait(sem, len(peers))

    return pl.pallas_call(
        body,
        out_shape=jax.ShapeDtypeStruct((B, SQ, H, D), jnp.float32),
        in_specs=[
            pl.BlockSpec(memory_space=pltpu.VMEM),
            pl.BlockSpec(memory_space=pl.ANY),
            pl.BlockSpec(memory_space=pl.ANY),
        ],
        out_specs=pl.BlockSpec(memory_space=pltpu.VMEM),
        scratch_shapes=[
            pltpu.VMEM((NSLOT, H, HALF, D), jnp.float32),
            pltpu.VMEM((NSLOT, H, HALF, D), jnp.float32),
            pltpu.VMEM((B, SQ, H, D), jnp.float32),
            pltpu.VMEM((3, B, SQ, H, D), jnp.float32),
            pltpu.VMEM((B, 2, SQ, H), jnp.float32),
            pltpu.VMEM((3, B, 2, SQ, H), jnp.float32),
            pltpu.SemaphoreType.DMA((NSLOT, 2, H)),
            pltpu.SemaphoreType.DMA((3, B, 2)),
            pltpu.SemaphoreType.DMA((3, B, 2)),
        ],
        compiler_params=pltpu.CompilerParams(
            collective_id=0, vmem_limit_bytes=64 * 1024 * 1024),
    )(Q, K, V)


# device time: 52153 ns/iter; 1.2376x vs baseline; 1.2376x over previous
import functools
import os

import jax
import jax.numpy as jnp
from jax import lax
from jax.experimental import pallas as pl
from jax.experimental.pallas import tpu as pltpu

B, SQ, H, D = 8, 8, 16, 128
SKV_LOCAL = 1024
HALF = SKV_LOCAL // 2
NSLOT = 3
SCALE = D ** -0.5

_KVAR = os.environ.get("KVAR", "full")


def kernel(Q, K, V):
    def body(q_ref, k_hbm, v_hbm, o_ref,
             k_buf, v_buf, o_acc, o_recv, st_acc, st_recv,
             copy_sems, send_sems, recv_sems):
        my_x = lax.axis_index("x")
        my_y = lax.axis_index("y")
        peers = ((my_x, 1 - my_y), (1 - my_x, my_y), (1 - my_x, 1 - my_y))

        barrier = pltpu.get_barrier_semaphore()
        for peer in peers:
            pl.semaphore_signal(barrier, inc=1, device_id=peer,
                                device_id_type=pl.DeviceIdType.MESH)
        pl.semaphore_wait(barrier, len(peers))

        row0 = my_x * HALF

        def dma_batch(b, slot):
            cps = []
            for h in range(H):
                cps.append(pltpu.make_async_copy(
                    k_hbm.at[b, pl.ds(row0, HALF), h],
                    k_buf.at[slot, h], copy_sems.at[slot, 0, h]))
                cps.append(pltpu.make_async_copy(
                    v_hbm.at[b, pl.ds(row0, HALF), h],
                    v_buf.at[slot, h], copy_sems.at[slot, 1, h]))
            for c in cps:
                c.start()
            return cps

        def exchange(phase, b):
            o_rdma = pltpu.make_async_remote_copy(
                src_ref=o_acc.at[b], dst_ref=o_recv.at[phase, b],
                send_sem=send_sems.at[phase, b, 0],
                recv_sem=recv_sems.at[phase, b, 0],
                device_id=peers[phase], device_id_type=pl.DeviceIdType.MESH)
            st_rdma = pltpu.make_async_remote_copy(
                src_ref=st_acc.at[b], dst_ref=st_recv.at[phase, b],
                send_sem=send_sems.at[phase, b, 1],
                recv_sem=recv_sems.at[phase, b, 1],
                device_id=peers[phase], device_id_type=pl.DeviceIdType.MESH)
            o_rdma.start()
            st_rdma.start()
            return (o_rdma, st_rdma)

        def combine_all(b, rdma_sets):
            for rdmas in rdma_sets:
                for r in rdmas:
                    r.wait()
            m_n = st_acc[b, 0]
            for p in range(3):
                m_n = jnp.maximum(m_n, st_recv[p, b, 0])
            a = jnp.exp(st_acc[b, 0] - m_n)
            l_n = a * st_acc[b, 1]
            o_n = a[..., None] * o_acc[b]
            for p in range(3):
                w = jnp.exp(st_recv[p, b, 0] - m_n)
                l_n = l_n + w * st_recv[p, b, 1]
                o_n = o_n + w[..., None] * o_recv[p, b]
            o_ref[b] = o_n / l_n[..., None]

        do_comm = _KVAR != "nocomm"
        inflight = {}

        pend = {0: dma_batch(0, 0), 1: dma_batch(1, 1)}
        for b in range(B):
            for c in pend.pop(b):
                c.wait()
            if b + 2 < B:
                pend[b + 2] = dma_batch(b + 2, (b + 2) % NSLOT)
            slot = b % NSLOT
            ms = []
            ls = []
            for h in range(H if _KVAR != "nocompute" else 0):
                q_h = q_ref[b, :, h, :]
                k_h = k_buf[slot, h]
                v_h = v_buf[slot, h]
                s = lax.dot_general(
                    q_h, k_h, (((1,), (1,)), ((), ())),
                    preferred_element_type=jnp.float32) * SCALE
                m = jnp.max(s, axis=-1)
                p = jnp.exp(s - m[:, None])
                l = jnp.sum(p, axis=-1)
                o_h = lax.dot_general(
                    p, v_h, (((1,), (0,)), ((), ())),
                    preferred_element_type=jnp.float32)
                o_acc[b, :, h, :] = o_h
                ms.append(m)
                ls.append(l)
            if ms:
                st_acc[b, 0] = jnp.stack(ms, axis=1)
                st_acc[b, 1] = jnp.stack(ls, axis=1)

            if do_comm:
                inflight[b] = [exchange(p, b) for p in range(3)]
                if b >= 1:
                    combine_all(b - 1, inflight.pop(b - 1))

        if do_comm:
            combine_all(B - 1, inflight.pop(B - 1))
        else:
            for b in range(B):
                o_ref[b] = o_acc[b] / st_acc[b, 1][..., None]

        @functools.partial(pl.run_scoped, sem=pltpu.SemaphoreType.REGULAR)
        def _(sem):
            for peer in peers:
                pl.semaphore_signal(sem, inc=1, device_id=peer,
                                    device_id_type=pl.DeviceIdType.MESH)
            pl.semaphore_wait(sem, len(peers))

    return pl.pallas_call(
        body,
        out_shape=jax.ShapeDtypeStruct((B, SQ, H, D), jnp.float32),
        in_specs=[
            pl.BlockSpec(memory_space=pltpu.VMEM),
            pl.BlockSpec(memory_space=pl.ANY),
            pl.BlockSpec(memory_space=pl.ANY),
        ],
        out_specs=pl.BlockSpec(memory_space=pltpu.VMEM),
        scratch_shapes=[
            pltpu.VMEM((NSLOT, H, HALF, D), jnp.float32),
            pltpu.VMEM((NSLOT, H, HALF, D), jnp.float32),
            pltpu.VMEM((B, SQ, H, D), jnp.float32),
            pltpu.VMEM((3, B, SQ, H, D), jnp.float32),
            pltpu.VMEM((B, 2, SQ, H), jnp.float32),
            pltpu.VMEM((3, B, 2, SQ, H), jnp.float32),
            pltpu.SemaphoreType.DMA((NSLOT, 2, H)),
            pltpu.SemaphoreType.DMA((3, B, 2)),
            pltpu.SemaphoreType.DMA((3, B, 2)),
        ],
        compiler_params=pltpu.CompilerParams(
            collective_id=0, vmem_limit_bytes=64 * 1024 * 1024),
    )(Q, K, V)


# device time: 52149 ns/iter; 1.2377x vs baseline; 1.0001x over previous
import functools
import os

import jax
import jax.numpy as jnp
from jax import lax
from jax.experimental import pallas as pl
from jax.experimental.pallas import tpu as pltpu

B, SQ, H, D = 8, 8, 16, 128
SKV_LOCAL = 1024
HALF = SKV_LOCAL // 2
NSLOT = 4
SCALE = D ** -0.5

_KVAR = os.environ.get("KVAR", "full")


def kernel(Q, K, V):
    def body(q_ref, k_hbm, v_hbm, o_ref,
             k_buf, v_buf, o_acc, o_recv, st_acc, st_recv,
             copy_sems, send_sems, recv_sems):
        my_x = lax.axis_index("x")
        my_y = lax.axis_index("y")
        peers = ((my_x, 1 - my_y), (1 - my_x, my_y), (1 - my_x, 1 - my_y))

        barrier = pltpu.get_barrier_semaphore()
        for peer in peers:
            pl.semaphore_signal(barrier, inc=1, device_id=peer,
                                device_id_type=pl.DeviceIdType.MESH)
        pl.semaphore_wait(barrier, len(peers))

        row0 = my_x * HALF

        def dma_batch(b, slot):
            cps = []
            for h in range(H):
                cps.append(pltpu.make_async_copy(
                    k_hbm.at[b, pl.ds(row0, HALF), h],
                    k_buf.at[slot, h], copy_sems.at[slot, 0, h]))
                cps.append(pltpu.make_async_copy(
                    v_hbm.at[b, pl.ds(row0, HALF), h],
                    v_buf.at[slot, h], copy_sems.at[slot, 1, h]))
            for c in cps:
                c.start()
            return cps

        def exchange(phase, b):
            o_rdma = pltpu.make_async_remote_copy(
                src_ref=o_acc.at[b], dst_ref=o_recv.at[phase, b],
                send_sem=send_sems.at[phase, b, 0],
                recv_sem=recv_sems.at[phase, b, 0],
                device_id=peers[phase], device_id_type=pl.DeviceIdType.MESH)
            st_rdma = pltpu.make_async_remote_copy(
                src_ref=st_acc.at[b], dst_ref=st_recv.at[phase, b],
                send_sem=send_sems.at[phase, b, 1],
                recv_sem=recv_sems.at[phase, b, 1],
                device_id=peers[phase], device_id_type=pl.DeviceIdType.MESH)
            o_rdma.start()
            st_rdma.start()
            return (o_rdma, st_rdma)

        def combine_all(b, rdma_sets):
            for rdmas in rdma_sets:
                for r in rdmas:
                    r.wait()
            m_n = st_acc[b, 0]
            for p in range(3):
                m_n = jnp.maximum(m_n, st_recv[p, b, 0])
            a = jnp.exp(st_acc[b, 0] - m_n)
            l_n = a * st_acc[b, 1]
            o_n = a[..., None] * o_acc[b]
            for p in range(3):
                w = jnp.exp(st_recv[p, b, 0] - m_n)
                l_n = l_n + w * st_recv[p, b, 1]
                o_n = o_n + w[..., None] * o_recv[p, b]
            o_ref[b] = o_n / l_n[..., None]

        do_comm = _KVAR != "nocomm"
        inflight = {}

        pend = {i: dma_batch(i, i) for i in range(NSLOT - 1)}
        for b in range(B):
            for c in pend.pop(b):
                c.wait()
            if b + NSLOT - 1 < B:
                pend[b + NSLOT - 1] = dma_batch(
                    b + NSLOT - 1, (b + NSLOT - 1) % NSLOT)
            slot = b % NSLOT
            ms = []
            ls = []
            for h in range(H if _KVAR != "nocompute" else 0):
                q_h = q_ref[b, :, h, :]
                k_h = k_buf[slot, h]
                v_h = v_buf[slot, h]
                s = lax.dot_general(
                    q_h, k_h, (((1,), (1,)), ((), ())),
                    preferred_element_type=jnp.float32) * SCALE
                m = jnp.max(s, axis=-1)
                p = jnp.exp(s - m[:, None])
                l = jnp.sum(p, axis=-1)
                o_h = lax.dot_general(
                    p, v_h, (((1,), (0,)), ((), ())),
                    preferred_element_type=jnp.float32)
                o_acc[b, :, h, :] = o_h
                ms.append(m)
                ls.append(l)
            if ms:
                st_acc[b, 0] = jnp.stack(ms, axis=1)
                st_acc[b, 1] = jnp.stack(ls, axis=1)

            if do_comm:
                inflight[b] = [exchange(p, b) for p in range(3)]
                if b >= 1:
                    combine_all(b - 1, inflight.pop(b - 1))

        if do_comm:
            combine_all(B - 1, inflight.pop(B - 1))
        else:
            for b in range(B):
                o_ref[b] = o_acc[b] / st_acc[b, 1][..., None]

        @functools.partial(pl.run_scoped, sem=pltpu.SemaphoreType.REGULAR)
        def _(sem):
            for peer in peers:
                pl.semaphore_signal(sem, inc=1, device_id=peer,
                                    device_id_type=pl.DeviceIdType.MESH)
            pl.semaphore_wait(sem, len(peers))

    return pl.pallas_call(
        body,
        out_shape=jax.ShapeDtypeStruct((B, SQ, H, D), jnp.float32),
        in_specs=[
            pl.BlockSpec(memory_space=pltpu.VMEM),
            pl.BlockSpec(memory_space=pl.ANY),
            pl.BlockSpec(memory_space=pl.ANY),
        ],
        out_specs=pl.BlockSpec(memory_space=pltpu.VMEM),
        scratch_shapes=[
            pltpu.VMEM((NSLOT, H, HALF, D), jnp.float32),
            pltpu.VMEM((NSLOT, H, HALF, D), jnp.float32),
            pltpu.VMEM((B, SQ, H, D), jnp.float32),
            pltpu.VMEM((3, B, SQ, H, D), jnp.float32),
            pltpu.VMEM((B, 2, SQ, H), jnp.float32),
            pltpu.VMEM((3, B, 2, SQ, H), jnp.float32),
            pltpu.SemaphoreType.DMA((NSLOT, 2, H)),
            pltpu.SemaphoreType.DMA((3, B, 2)),
            pltpu.SemaphoreType.DMA((3, B, 2)),
        ],
        compiler_params=pltpu.CompilerParams(
            collective_id=0, vmem_limit_bytes=64 * 1024 * 1024),
    )(Q, K, V)


# device time: 50576 ns/iter; 1.2762x vs baseline; 1.0311x over previous
import functools
import os

import jax
import jax.numpy as jnp
from jax import lax
from jax.experimental import pallas as pl
from jax.experimental.pallas import tpu as pltpu

B, SQ, H, D = 8, 8, 16, 128
SKV_LOCAL = 1024
HALF = SKV_LOCAL // 2
NSLOT = 4
SCALE = D ** -0.5

_KVAR = os.environ.get("KVAR", "full")


def kernel(Q, K, V):
    def body(q_ref, k_hbm, v_hbm, o_ref,
             k_buf, v_buf, o_acc, o_recv, st_acc, st_recv,
             copy_sems, send_sems, recv_sems):
        my_x = lax.axis_index("x")
        my_y = lax.axis_index("y")
        peers = ((my_x, 1 - my_y), (1 - my_x, my_y), (1 - my_x, 1 - my_y))

        row0 = my_x * HALF

        def dma_batch(b, slot):
            cps = []
            for h in range(H):
                cps.append(pltpu.make_async_copy(
                    k_hbm.at[b, pl.ds(row0, HALF), h],
                    k_buf.at[slot, h], copy_sems.at[slot, 0, h]))
                cps.append(pltpu.make_async_copy(
                    v_hbm.at[b, pl.ds(row0, HALF), h],
                    v_buf.at[slot, h], copy_sems.at[slot, 1, h]))
            for c in cps:
                c.start()
            return cps

        def exchange(phase, b):
            o_rdma = pltpu.make_async_remote_copy(
                src_ref=o_acc.at[b], dst_ref=o_recv.at[phase, b],
                send_sem=send_sems.at[phase, b, 0],
                recv_sem=recv_sems.at[phase, b, 0],
                device_id=peers[phase], device_id_type=pl.DeviceIdType.MESH)
            st_rdma = pltpu.make_async_remote_copy(
                src_ref=st_acc.at[b], dst_ref=st_recv.at[phase, b],
                send_sem=send_sems.at[phase, b, 1],
                recv_sem=recv_sems.at[phase, b, 1],
                device_id=peers[phase], device_id_type=pl.DeviceIdType.MESH)
            o_rdma.start()
            st_rdma.start()
            return (o_rdma, st_rdma)

        def combine_all(b, rdma_sets):
            for rdmas in rdma_sets:
                for r in rdmas:
                    r.wait()
            m_n = st_acc[b, 0]
            for p in range(3):
                m_n = jnp.maximum(m_n, st_recv[p, b, 0])
            a = jnp.exp(st_acc[b, 0] - m_n)
            l_n = a * st_acc[b, 1]
            o_n = a[..., None] * o_acc[b]
            for p in range(3):
                w = jnp.exp(st_recv[p, b, 0] - m_n)
                l_n = l_n + w * st_recv[p, b, 1]
                o_n = o_n + w[..., None] * o_recv[p, b]
            o_ref[b] = o_n / l_n[..., None]

        do_comm = _KVAR != "nocomm"
        inflight = {}

        pend = {i: dma_batch(i, i) for i in range(NSLOT - 1)}

        barrier = pltpu.get_barrier_semaphore()
        for peer in peers:
            pl.semaphore_signal(barrier, inc=1, device_id=peer,
                                device_id_type=pl.DeviceIdType.MESH)
        pl.semaphore_wait(barrier, len(peers))

        for b in range(B):
            for c in pend.pop(b):
                c.wait()
            if b + NSLOT - 1 < B:
                pend[b + NSLOT - 1] = dma_batch(
                    b + NSLOT - 1, (b + NSLOT - 1) % NSLOT)
            slot = b % NSLOT
            ms = []
            ls = []
            for h in range(H if _KVAR != "nocompute" else 0):
                q_h = q_ref[b, :, h, :]
                k_h = k_buf[slot, h]
                v_h = v_buf[slot, h]
                s = lax.dot_general(
                    q_h, k_h, (((1,), (1,)), ((), ())),
                    preferred_element_type=jnp.float32) * SCALE
                m = jnp.max(s, axis=-1)
                p = jnp.exp(s - m[:, None])
                l = jnp.sum(p, axis=-1)
                o_h = lax.dot_general(
                    p, v_h, (((1,), (0,)), ((), ())),
                    preferred_element_type=jnp.float32)
                o_acc[b, :, h, :] = o_h
                ms.append(m)
                ls.append(l)
            if ms:
                st_acc[b, 0] = jnp.stack(ms, axis=1)
                st_acc[b, 1] = jnp.stack(ls, axis=1)

            if do_comm:
                inflight[b] = [exchange(p, b) for p in range(3)]
                if b >= 1:
                    combine_all(b - 1, inflight.pop(b - 1))

        if do_comm:
            combine_all(B - 1, inflight.pop(B - 1))
        else:
            for b in range(B):
                o_ref[b] = o_acc[b] / st_acc[b, 1][..., None]

        @functools.partial(pl.run_scoped, sem=pltpu.SemaphoreType.REGULAR)
        def _(sem):
            for peer in peers:
                pl.semaphore_signal(sem, inc=1, device_id=peer,
                                    device_id_type=pl.DeviceIdType.MESH)
            pl.semaphore_wait(sem, len(peers))

    return pl.pallas_call(
        body,
        out_shape=jax.ShapeDtypeStruct((B, SQ, H, D), jnp.float32),
        in_specs=[
            pl.BlockSpec(memory_space=pltpu.VMEM),
            pl.BlockSpec(memory_space=pl.ANY),
            pl.BlockSpec(memory_space=pl.ANY),
        ],
        out_specs=pl.BlockSpec(memory_space=pltpu.VMEM),
        scratch_shapes=[
            pltpu.VMEM((NSLOT, H, HALF, D), jnp.float32),
            pltpu.VMEM((NSLOT, H, HALF, D), jnp.float32),
            pltpu.VMEM((B, SQ, H, D), jnp.float32),
            pltpu.VMEM((3, B, SQ, H, D), jnp.float32),
            pltpu.VMEM((B, 2, SQ, H), jnp.float32),
            pltpu.VMEM((3, B, 2, SQ, H), jnp.float32),
            pltpu.SemaphoreType.DMA((NSLOT, 2, H)),
            pltpu.SemaphoreType.DMA((3, B, 2)),
            pltpu.SemaphoreType.DMA((3, B, 2)),
        ],
        compiler_params=pltpu.CompilerParams(
            collective_id=0, vmem_limit_bytes=64 * 1024 * 1024),
    )(Q, K, V)
